# baseline (speedup 1.0000x reference)
import functools

import jax
import jax.numpy as jnp
from jax import lax
from jax.experimental import pallas as pl
from jax.experimental.pallas import tpu as pltpu
from jax.experimental.pallas import tpu_sc as plsc

B, D, L, H = 2048, 1024, 768, 8
BLK = 256
NBLK = B // BLK
SCALE = 0.0159
B2 = 2 * B


def _prep_body(sct, tct, sca, tca, pos_ref, offs_ref):
    r = lax.broadcasted_iota(jnp.int32, (B, B), 0)
    c = lax.broadcasted_iota(jnp.int32, (B, B), 1)
    tril = (r >= c).astype(jnp.bfloat16)
    col8 = lax.broadcasted_iota(jnp.int32, (B, H), 1)
    ones_row = jnp.ones((1, B), jnp.bfloat16)

    for k, ctx_ref in enumerate((sct, sca, tct, tca)):
        ids = jnp.argmax(ctx_ref[...], axis=1).astype(jnp.int32)
        m = (col8 == ids[:, None]).astype(jnp.bfloat16)
        rank = lax.dot_general(tril, m, (((1,), (0,)), ((), ())),
                               preferred_element_type=jnp.float32)
        counts = lax.dot_general(ones_row, m, (((1,), (0,)), ((), ())),
                                 preferred_element_type=jnp.float32)
        cols = [jnp.zeros((1, 1), jnp.float32)]
        acc = jnp.zeros((1, 1), jnp.float32)
        for h in range(1, H):
            acc = acc + counts[:, h - 1:h]
            cols.append(acc)
        offs = jnp.concatenate(cols, axis=1)
        slot = jnp.sum(m.astype(jnp.float32) * (offs + rank - 1.0),
                       axis=1, keepdims=True)
        pos_ref[:, k:k + 1] = slot.astype(jnp.int32)
        offs_ref[k] = offs.astype(jnp.int32)


def _prep(sct, tct, sca, tca):
    return pl.pallas_call(
        _prep_body,
        grid=(1,),
        in_specs=[pl.BlockSpec((B, H), lambda i: (0, 0))] * 4,
        out_specs=[pl.BlockSpec((B, 4), lambda i: (0, 0)),
                   pl.BlockSpec((4, 1, H), lambda i: (0, 0, 0))],
        out_shape=[jax.ShapeDtypeStruct((B, 4), jnp.int32),
                   jax.ShapeDtypeStruct((4, 1, H), jnp.int32)],
    )(sct, tct, sca, tca)


def _tc1_body(x_ref, wb, wet, wea, base_ref, sht_ref, sha_ref):
    xb = x_ref[...].astype(jnp.bfloat16)
    h_base = lax.dot_general(xb, wb[...], (((1,), (1,)), ((), ())),
                             preferred_element_type=jnp.float32)
    base_ref[...] = lax.dot_general(h_base.astype(jnp.bfloat16), wb[...],
                                    (((1,), (0,)), ((), ())),
                                    preferred_element_type=jnp.float32)
    sht_ref[...] = lax.dot_general(xb, wet[...], (((1,), (1,)), ((), ())),
                                   preferred_element_type=jnp.float32)
    sha_ref[...] = lax.dot_general(xb, wea[...], (((1,), (1,)), ((), ())),
                                   preferred_element_type=jnp.float32)


def _tc1(expr, wb, wet, wea):
    row = lambda i: (i, 0)
    full = lambda i: (0, 0)
    return pl.pallas_call(
        _tc1_body,
        grid=(NBLK,),
        in_specs=[pl.BlockSpec((BLK, D), row),
                  pl.BlockSpec((L, D), full),
                  pl.BlockSpec((L, D), full),
                  pl.BlockSpec((L, D), full)],
        out_specs=[pl.BlockSpec((BLK, D), row),
                   pl.BlockSpec((BLK, L), row),
                   pl.BlockSpec((BLK, L), row)],
        out_shape=[jax.ShapeDtypeStruct((B, D), jnp.float32),
                   jax.ShapeDtypeStruct((B, L), jnp.float32),
                   jax.ShapeDtypeStruct((B, L), jnp.float32)],
    )(expr, wb, wet, wea)


_MESH = plsc.VectorSubcoreMesh(core_axis_name="c", subcore_axis_name="s")
_NW = 32
_CH1 = B // _NW
_CH2 = B2 // _NW


def _wid():
    return lax.axis_index("s") * 2 + lax.axis_index("c")


@functools.partial(
    pl.kernel, mesh=_MESH,
    out_type=jax.ShapeDtypeStruct((B2, L), jnp.float32),
    scratch_types=[pltpu.VMEM((_CH1,), jnp.int32),
                   pltpu.VMEM((_CH1, L), jnp.float32),
                   pltpu.SemaphoreType.DMA],
)
def _sc_scatter(sh_t, sh_a, p1, out, idx_v, rows_v, sem):
    base = _wid() * _CH1
    pltpu.sync_copy(p1.at[pl.ds(base, _CH1)], idx_v)
    pltpu.sync_copy(sh_t.at[pl.ds(base, _CH1)], rows_v)
    pltpu.async_copy(rows_v, out.at[idx_v], sem).wait()
    pltpu.sync_copy(p1.at[pl.ds(B + base, _CH1)], idx_v)
    pltpu.sync_copy(sh_a.at[pl.ds(base, _CH1)], rows_v)
    pltpu.async_copy(rows_v, out.at[idx_v], sem).wait()


@functools.partial(
    pl.kernel, mesh=_MESH,
    out_type=jax.ShapeDtypeStruct((B2, L), jnp.float32),
    scratch_types=[pltpu.VMEM((_CH2,), jnp.int32),
                   pltpu.VMEM((_CH2,), jnp.int32),
                   pltpu.VMEM((_CH2, L), jnp.float32),
                   pltpu.SemaphoreType.DMA],
)
def _sc_resort(src, p1, p2, out, idx1_v, idx2_v, rows_v, sem):
    base = _wid() * _CH2
    pltpu.sync_copy(p1.at[pl.ds(base, _CH2)], idx1_v)
    pltpu.sync_copy(p2.at[pl.ds(base, _CH2)], idx2_v)
    pltpu.async_copy(src.at[idx1_v], rows_v, sem).wait()
    pltpu.async_copy(rows_v, out.at[idx2_v], sem).wait()


@functools.partial(
    pl.kernel, mesh=_MESH,
    out_type=jax.ShapeDtypeStruct((B2, L), jnp.float32),
    scratch_types=[pltpu.VMEM((_CH2,), jnp.int32),
                   pltpu.VMEM((_CH2, L), jnp.float32),
                   pltpu.SemaphoreType.DMA],
)
def _sc_unsort(src, p2, out, idx_v, rows_v, sem):
    base = _wid() * _CH2
    pltpu.sync_copy(p2.at[pl.ds(base, _CH2)], idx_v)
    pltpu.async_copy(src.at[idx_v], rows_v, sem).wait()
    pltpu.sync_copy(rows_v, out.at[pl.ds(base, _CH2)])


def _grouped_body(x_ref, wh_ref, offs_ref, o_ref):
    i = pl.program_id(0)
    s0 = (i % NBLK) * BLK
    slots = lax.broadcasted_iota(jnp.int32, (BLK, H), 0) + s0
    ge = (slots >= offs_ref[0]).astype(jnp.int32)
    id_col = jnp.sum(ge, axis=1, keepdims=True) - 1
    lo = jnp.min(id_col)
    hi = jnp.max(id_col)
    xb = x_ref[...].astype(jnp.bfloat16)
    o_ref[...] = jnp.zeros((BLK, L), jnp.float32)
    for h in range(H):
        @pl.when((lo <= h) & (h <= hi))
        def _():
            p = lax.dot_general(xb, wh_ref[0, h], (((1,), (1,)), ((), ())),
                                preferred_element_type=jnp.float32)
            o_ref[...] += jnp.where(id_col == h, p, 0.0)


def _grouped(x_sorted, whs, offs, offs_base):
    row = lambda i: (i, 0)
    return pl.pallas_call(
        _grouped_body,
        grid=(2 * NBLK,),
        in_specs=[pl.BlockSpec((BLK, L), row),
                  pl.BlockSpec((1, H, L, L), lambda i: (i // NBLK, 0, 0, 0)),
                  pl.BlockSpec((1, 1, H), lambda i: (offs_base + i // NBLK, 0, 0))],
        out_specs=pl.BlockSpec((BLK, L), row),
        out_shape=jax.ShapeDtypeStruct((B2, L), jnp.float32),
    )(x_sorted, whs, offs)


def _tc4_body(base_ref, dt_ref, da_ref, wdt, wda, o_ref):
    ct = lax.dot_general(dt_ref[...].astype(jnp.bfloat16), wdt[...],
                         (((1,), (1,)), ((), ())),
                         preferred_element_type=jnp.float32)
    ca = lax.dot_general(da_ref[...].astype(jnp.bfloat16), wda[...],
                         (((1,), (1,)), ((), ())),
                         preferred_element_type=jnp.float32)
    o_ref[...] = base_ref[...] + SCALE * ct + SCALE * ca


def _tc4(out_base, dec_tokens, wdt, wda):
    row = lambda i: (i, 0)
    full = lambda i: (0, 0)
    return pl.pallas_call(
        _tc4_body,
        grid=(NBLK,),
        in_specs=[pl.BlockSpec((BLK, D), row),
                  pl.BlockSpec((BLK, L), row),
                  pl.BlockSpec((BLK, L), lambda i: (i + NBLK, 0)),
                  pl.BlockSpec((D, L), full),
                  pl.BlockSpec((D, L), full)],
        out_specs=pl.BlockSpec((BLK, D), row),
        out_shape=jax.ShapeDtypeStruct((B, D), jnp.float32),
    )(out_base, dec_tokens, dec_tokens, wdt, wda)


def kernel(expr, src_ctx_tissue, tgt_ctx_tissue, src_ctx_assay, tgt_ctx_assay,
           W_base, W_enc_tissue, W_dec_tissue, W_heads_tissue,
           W_enc_assay, W_dec_assay, W_heads_assay):
    bf = jnp.bfloat16
    wb = W_base.astype(bf)
    wet = W_enc_tissue.astype(bf)
    wea = W_enc_assay.astype(bf)
    wdt = W_dec_tissue.astype(bf)
    wda = W_dec_assay.astype(bf)
    whs = jnp.stack([W_heads_tissue, W_heads_assay]).astype(bf)

    pos4, offs = _prep(src_ctx_tissue, tgt_ctx_tissue,
                       src_ctx_assay, tgt_ctx_assay)
    p1 = jnp.concatenate([pos4[:, 0], pos4[:, 1] + B])
    p2 = jnp.concatenate([pos4[:, 2], pos4[:, 3] + B])

    out_base, sh_t, sh_a = _tc1(expr, wb, wet, wea)
    sorted_src = _sc_scatter(sh_t, sh_a, p1)
    routed1 = _grouped(sorted_src, whs, offs, 0)
    sorted_tgt = _sc_resort(routed1, p1, p2)
    routed2 = _grouped(sorted_tgt, whs, offs, 2)
    dec_tokens = _sc_unsort(routed2, p2)
    return _tc4(out_base, dec_tokens, wdt, wda)


# speedup vs baseline: 1.0448x; 1.0448x over previous
import functools

import jax
import jax.numpy as jnp
from jax import lax
from jax.experimental import pallas as pl
from jax.experimental.pallas import tpu as pltpu
from jax.experimental.pallas import tpu_sc as plsc

B, D, L, H = 2048, 1024, 768, 8
BLK = 256
NBLK = B // BLK
SCALE = 0.0159


def _prep_body(sct, tct, sca, tca, pos_ref, offs_ref):
    r = lax.broadcasted_iota(jnp.int32, (B, B), 0)
    c = lax.broadcasted_iota(jnp.int32, (B, B), 1)
    tril = (r >= c).astype(jnp.bfloat16)
    col8 = lax.broadcasted_iota(jnp.int32, (B, H), 1)
    ones_row = jnp.ones((1, B), jnp.bfloat16)

    for k, ctx_ref in enumerate((sct, sca, tct, tca)):
        ids = jnp.argmax(ctx_ref[...], axis=1).astype(jnp.int32)
        m = (col8 == ids[:, None]).astype(jnp.bfloat16)
        rank = lax.dot_general(tril, m, (((1,), (0,)), ((), ())),
                               preferred_element_type=jnp.float32)
        counts = lax.dot_general(ones_row, m, (((1,), (0,)), ((), ())),
                                 preferred_element_type=jnp.float32)
        cols = [jnp.zeros((1, 1), jnp.float32)]
        acc = jnp.zeros((1, 1), jnp.float32)
        for h in range(1, H):
            acc = acc + counts[:, h - 1:h]
            cols.append(acc)
        offs = jnp.concatenate(cols, axis=1)
        slot = jnp.sum(m.astype(jnp.float32) * (offs + rank - 1.0),
                       axis=1, keepdims=True)
        pos_ref[:, k:k + 1] = slot.astype(jnp.int32)
        offs_ref[k] = offs.astype(jnp.int32)


def _prep(sct, tct, sca, tca):
    return pl.pallas_call(
        _prep_body,
        grid=(1,),
        in_specs=[pl.BlockSpec((B, H), lambda i: (0, 0))] * 4,
        out_specs=[pl.BlockSpec((B, 4), lambda i: (0, 0)),
                   pl.BlockSpec((4, 1, H), lambda i: (0, 0, 0))],
        out_shape=[jax.ShapeDtypeStruct((B, 4), jnp.int32),
                   jax.ShapeDtypeStruct((4, 1, H), jnp.int32)],
    )(sct, tct, sca, tca)


def _tc1_body(x_ref, wb, wet, wea, base_ref, sht_ref, sha_ref):
    xb = x_ref[...].astype(jnp.bfloat16)
    h_base = lax.dot_general(xb, wb[...], (((1,), (1,)), ((), ())),
                             preferred_element_type=jnp.float32)
    base_ref[...] = lax.dot_general(h_base.astype(jnp.bfloat16), wb[...],
                                    (((1,), (0,)), ((), ())),
                                    preferred_element_type=jnp.float32)
    sht_ref[...] = lax.dot_general(xb, wet[...], (((1,), (1,)), ((), ())),
                                   preferred_element_type=jnp.float32)
    sha_ref[...] = lax.dot_general(xb, wea[...], (((1,), (1,)), ((), ())),
                                   preferred_element_type=jnp.float32)


def _tc1(expr, wb, wet, wea):
    row = lambda i: (i, 0)
    full = lambda i: (0, 0)
    return pl.pallas_call(
        _tc1_body,
        grid=(NBLK,),
        in_specs=[pl.BlockSpec((BLK, D), row),
                  pl.BlockSpec((L, D), full),
                  pl.BlockSpec((L, D), full),
                  pl.BlockSpec((L, D), full)],
        out_specs=[pl.BlockSpec((BLK, D), row),
                   pl.BlockSpec((BLK, L), row),
                   pl.BlockSpec((BLK, L), row)],
        out_shape=[jax.ShapeDtypeStruct((B, D), jnp.float32),
                   jax.ShapeDtypeStruct((B, L), jnp.float32),
                   jax.ShapeDtypeStruct((B, L), jnp.float32)],
    )(expr, wb, wet, wea)


_MESH = plsc.VectorSubcoreMesh(core_axis_name="c", subcore_axis_name="s")
_NW = 32
_CH = B // _NW


def _wid():
    return lax.axis_index("s") * 2 + lax.axis_index("c")


@functools.partial(
    pl.kernel, mesh=_MESH,
    out_type=jax.ShapeDtypeStruct((B, L), jnp.float32),
    scratch_types=[pltpu.VMEM((_CH,), jnp.int32),
                   pltpu.VMEM((_CH, L), jnp.float32),
                   pltpu.SemaphoreType.DMA],
)
def _sc_sort(src, p1, out, idx_v, rows_v, sem):
    base = _wid() * _CH
    pltpu.sync_copy(p1.at[pl.ds(base, _CH)], idx_v)
    pltpu.sync_copy(src.at[pl.ds(base, _CH)], rows_v)
    pltpu.async_copy(rows_v, out.at[idx_v], sem).wait()


@functools.partial(
    pl.kernel, mesh=_MESH,
    out_type=jax.ShapeDtypeStruct((B, L), jnp.float32),
    scratch_types=[pltpu.VMEM((_CH,), jnp.int32),
                   pltpu.VMEM((_CH,), jnp.int32),
                   pltpu.VMEM((_CH, L), jnp.float32),
                   pltpu.SemaphoreType.DMA],
)
def _sc_resort(src, p1, p2, out, idx1_v, idx2_v, rows_v, sem):
    base = _wid() * _CH
    pltpu.sync_copy(p1.at[pl.ds(base, _CH)], idx1_v)
    pltpu.sync_copy(p2.at[pl.ds(base, _CH)], idx2_v)
    pltpu.async_copy(src.at[idx1_v], rows_v, sem).wait()
    pltpu.async_copy(rows_v, out.at[idx2_v], sem).wait()


@functools.partial(
    pl.kernel, mesh=_MESH,
    out_type=jax.ShapeDtypeStruct((B, L), jnp.float32),
    scratch_types=[pltpu.VMEM((_CH,), jnp.int32),
                   pltpu.VMEM((_CH, L), jnp.float32),
                   pltpu.SemaphoreType.DMA],
)
def _sc_unsort(src, p2, out, idx_v, rows_v, sem):
    base = _wid() * _CH
    pltpu.sync_copy(p2.at[pl.ds(base, _CH)], idx_v)
    pltpu.async_copy(src.at[idx_v], rows_v, sem).wait()
    pltpu.sync_copy(rows_v, out.at[pl.ds(base, _CH)])


def _grouped_body(x_ref, wh_ref, offs_ref, o_ref):
    i = pl.program_id(0)
    s0 = i * BLK
    slots = lax.broadcasted_iota(jnp.int32, (BLK, H), 0) + s0
    ge = (slots >= offs_ref[0]).astype(jnp.int32)
    id_col = jnp.sum(ge, axis=1, keepdims=True) - 1
    lo = jnp.min(id_col)
    hi = jnp.max(id_col)
    xb = x_ref[...].astype(jnp.bfloat16)
    o_ref[...] = jnp.zeros((BLK, L), jnp.float32)
    for h in range(H):
        @pl.when((lo <= h) & (h <= hi))
        def _():
            p = lax.dot_general(xb, wh_ref[h], (((1,), (1,)), ((), ())),
                                preferred_element_type=jnp.float32)
            o_ref[...] += jnp.where(id_col == h, p, 0.0)


def _grouped(x_sorted, wh, offs, offs_row):
    row = lambda i: (i, 0)
    return pl.pallas_call(
        _grouped_body,
        grid=(NBLK,),
        in_specs=[pl.BlockSpec((BLK, L), row),
                  pl.BlockSpec((H, L, L), lambda i: (0, 0, 0)),
                  pl.BlockSpec((1, 1, H), lambda i: (offs_row, 0, 0))],
        out_specs=pl.BlockSpec((BLK, L), row),
        out_shape=jax.ShapeDtypeStruct((B, L), jnp.float32),
    )(x_sorted, wh, offs)


def _tc4_body(base_ref, dt_ref, da_ref, wdt, wda, o_ref):
    ct = lax.dot_general(dt_ref[...].astype(jnp.bfloat16), wdt[...],
                         (((1,), (1,)), ((), ())),
                         preferred_element_type=jnp.float32)
    ca = lax.dot_general(da_ref[...].astype(jnp.bfloat16), wda[...],
                         (((1,), (1,)), ((), ())),
                         preferred_element_type=jnp.float32)
    o_ref[...] = base_ref[...] + SCALE * ct + SCALE * ca


def _tc4(out_base, dec_t, dec_a, wdt, wda):
    row = lambda i: (i, 0)
    full = lambda i: (0, 0)
    return pl.pallas_call(
        _tc4_body,
        grid=(NBLK,),
        in_specs=[pl.BlockSpec((BLK, D), row),
                  pl.BlockSpec((BLK, L), row),
                  pl.BlockSpec((BLK, L), row),
                  pl.BlockSpec((D, L), full),
                  pl.BlockSpec((D, L), full)],
        out_specs=pl.BlockSpec((BLK, D), row),
        out_shape=jax.ShapeDtypeStruct((B, D), jnp.float32),
    )(out_base, dec_t, dec_a, wdt, wda)


def kernel(expr, src_ctx_tissue, tgt_ctx_tissue, src_ctx_assay, tgt_ctx_assay,
           W_base, W_enc_tissue, W_dec_tissue, W_heads_tissue,
           W_enc_assay, W_dec_assay, W_heads_assay):
    bf = jnp.bfloat16
    wb = W_base.astype(bf)
    wet = W_enc_tissue.astype(bf)
    wea = W_enc_assay.astype(bf)
    wdt = W_dec_tissue.astype(bf)
    wda = W_dec_assay.astype(bf)
    wht = W_heads_tissue.astype(bf)
    wha = W_heads_assay.astype(bf)

    pos4, offs = _prep(src_ctx_tissue, tgt_ctx_tissue,
                       src_ctx_assay, tgt_ctx_assay)
    p1_t, p1_a = pos4[:, 0], pos4[:, 1]
    p2_t, p2_a = pos4[:, 2], pos4[:, 3]

    out_base, sh_t, sh_a = _tc1(expr, wb, wet, wea)

    srt_t = _sc_sort(sh_t, p1_t)
    srt_a = _sc_sort(sh_a, p1_a)
    r1_t = _grouped(srt_t, wht, offs, 0)
    r1_a = _grouped(srt_a, wha, offs, 1)
    rs_t = _sc_resort(r1_t, p1_t, p2_t)
    rs_a = _sc_resort(r1_a, p1_a, p2_a)
    r2_t = _grouped(rs_t, wht, offs, 2)
    r2_a = _grouped(rs_a, wha, offs, 3)
    dec_t = _sc_unsort(r2_t, p2_t)
    dec_a = _sc_unsort(r2_a, p2_a)
    return _tc4(out_base, dec_t, dec_a, wdt, wda)
